# baseline (speedup 1.0000x reference)
import functools

import jax
import jax.numpy as jnp
from jax import lax
from jax.experimental import pallas as pl
from jax.experimental.pallas import tpu as pltpu
from jax.experimental.pallas import tpu_sc as plsc

_NC = 2
_NS = 16
_NW = _NC * _NS
_DEGW = 128
_K = 40


def _pad_rows(N):
    return (N // 128 + 1) * 128


_NBUF = 2
_NIDX = 4


def _build_deg(N, CH):
    NP = _pad_rows(N)
    ROWS = NP // _NS
    mesh = plsc.VectorSubcoreMesh(core_axis_name="c", subcore_axis_name="s")
    assert CH % _NIDX == 0

    @functools.partial(
        pl.kernel,
        out_type=jax.ShapeDtypeStruct((_NC * NP, _DEGW), jnp.float32),
        mesh=mesh,
        scratch_types=[
            [pltpu.VMEM((_K,), jnp.int32) for _ in range(_NIDX)],
            pltpu.VMEM((_K, _DEGW), jnp.float32),
            pltpu.SemaphoreType.DMA((_NIDX,)),
            pltpu.VMEM_SHARED((NP, _DEGW), jnp.float32),
        ],
    )
    def deg_k(dst_hbm, ones_hbm, zeros_hbm, out_hbm, idx_v, ones_v, dsem,
              acc_sh):
        c = lax.axis_index("c")
        s = lax.axis_index("s")
        w = c * _NS + s
        pltpu.sync_copy(ones_hbm, ones_v)
        pltpu.sync_copy(zeros_hbm, acc_sh.at[pl.ds(s * ROWS, ROWS)])
        for q in range(_NIDX):
            pltpu.async_copy(dst_hbm.at[w, q], idx_v[q], dsem.at[q])
        plsc.subcore_barrier()

        def body(j, carry):
            for r in range(_NIDX):
                i = j * _NIDX + r
                pltpu.make_async_copy(dst_hbm.at[0, 0], idx_v[r],
                                      dsem.at[r]).wait()
                pltpu.sync_copy(ones_v, acc_sh.at[idx_v[r]], add=True)

                @pl.when(i + _NIDX < CH)
                def _():
                    pltpu.async_copy(dst_hbm.at[w, i + _NIDX], idx_v[r],
                                     dsem.at[r])
            return carry

        lax.fori_loop(0, CH // _NIDX, body, 0)
        plsc.subcore_barrier()
        pltpu.sync_copy(acc_sh.at[pl.ds(s * ROWS, ROWS)],
                        out_hbm.at[pl.ds(c * NP + s * ROWS, ROWS)])

    return deg_k


def _build_agg(N, H, CH):
    NP = _pad_rows(N)
    ROWS = NP // _NS
    mesh = plsc.VectorSubcoreMesh(core_axis_name="c", subcore_axis_name="s")
    assert CH % _NIDX == 0

    @functools.partial(
        pl.kernel,
        out_type=jax.ShapeDtypeStruct((_NC * NP, H), jnp.float32),
        mesh=mesh,
        scratch_types=[
            [pltpu.VMEM((_K,), jnp.int32) for _ in range(_NIDX)],
            [pltpu.VMEM((_K,), jnp.int32) for _ in range(_NIDX)],
            [pltpu.VMEM((_K, H), jnp.float32) for _ in range(_NBUF)],
            pltpu.SemaphoreType.DMA((_NIDX,)),
            pltpu.SemaphoreType.DMA((_NIDX,)),
            pltpu.SemaphoreType.DMA((_NBUF,)),
            pltpu.VMEM_SHARED((NP, H), jnp.float32),
        ],
    )
    def agg_k(g_hbm, src_hbm, dst_hbm, zeros_hbm, out_hbm,
              isrc_v, idst_v, rows_v, isem, dsem, gsem, acc_sh):
        c = lax.axis_index("c")
        s = lax.axis_index("s")
        w = c * _NS + s
        pltpu.sync_copy(zeros_hbm, acc_sh.at[pl.ds(s * ROWS, ROWS)])

        for q in range(_NIDX):
            pltpu.async_copy(src_hbm.at[w, q], isrc_v[q], isem.at[q])
            pltpu.async_copy(dst_hbm.at[w, q], idst_v[q], dsem.at[q])
        plsc.subcore_barrier()
        for b in range(_NBUF):
            pltpu.make_async_copy(src_hbm.at[0, 0], isrc_v[b],
                                  isem.at[b]).wait()
            pltpu.async_copy(g_hbm.at[isrc_v[b]], rows_v[b], gsem.at[b])

        def outer(j, carry):
            for r in range(_NIDX):
                i = j * _NIDX + r
                b = r % _NBUF
                pltpu.make_async_copy(g_hbm.at[pl.ds(0, _K)], rows_v[b],
                                      gsem.at[b]).wait()
                pltpu.make_async_copy(src_hbm.at[0, 0], idst_v[r],
                                      dsem.at[r]).wait()
                pltpu.sync_copy(rows_v[b], acc_sh.at[idst_v[r]], add=True)

                @pl.when(i + _NBUF < CH)
                def _():
                    nxt = i + _NBUF
                    pltpu.make_async_copy(src_hbm.at[0, 0],
                                          isrc_v[(r + _NBUF) % _NIDX],
                                          isem.at[(r + _NBUF) % _NIDX]).wait()
                    pltpu.async_copy(g_hbm.at[isrc_v[(r + _NBUF) % _NIDX]],
                                     rows_v[b], gsem.at[b])

                @pl.when(i + _NIDX < CH)
                def _():
                    nn = i + _NIDX
                    pltpu.async_copy(src_hbm.at[w, nn], isrc_v[r],
                                     isem.at[r])
                    pltpu.async_copy(dst_hbm.at[w, nn], idst_v[r],
                                     dsem.at[r])
            return carry

        lax.fori_loop(0, CH // _NIDX, outer, 0)
        plsc.subcore_barrier()
        pltpu.sync_copy(acc_sh.at[pl.ds(s * ROWS, ROWS)],
                        out_hbm.at[pl.ds(c * NP + s * ROWS, ROWS)])

    return agg_k


def _mm_body(x_ref, w_ref, o_ref):
    o_ref[...] = jnp.dot(x_ref[...], w_ref[...],
                         preferred_element_type=jnp.float32)


def _matmul(x, w, rb=1000):
    n, f = x.shape
    h = w.shape[1]
    return pl.pallas_call(
        _mm_body,
        grid=(n // rb,),
        in_specs=[pl.BlockSpec((rb, f), lambda i: (i, 0)),
                  pl.BlockSpec((f, h), lambda i: (0, 0))],
        out_specs=pl.BlockSpec((rb, h), lambda i: (i, 0)),
        out_shape=jax.ShapeDtypeStruct((n, h), jnp.float32),
    )(x, w)


def _dinv_scale_body(deg_ref, hw_ref, dinv_ref, g_ref):
    deg = deg_ref[0, :, 0] + deg_ref[1, :, 0] + 1.0
    dinv = lax.rsqrt(deg)
    dinv_ref[...] = dinv[:, None]
    g_ref[...] = hw_ref[...] * dinv[:, None]


def _dinv_scale(deg2, hw, rb=1000):
    n, h = hw.shape
    return pl.pallas_call(
        _dinv_scale_body,
        grid=(n // rb,),
        in_specs=[pl.BlockSpec((2, rb, _DEGW), lambda i: (0, i, 0)),
                  pl.BlockSpec((rb, h), lambda i: (i, 0))],
        out_specs=[pl.BlockSpec((rb, 1), lambda i: (i, 0)),
                   pl.BlockSpec((rb, h), lambda i: (i, 0))],
        out_shape=[jax.ShapeDtypeStruct((n, 1), jnp.float32),
                   jax.ShapeDtypeStruct((n, h), jnp.float32)],
    )(deg2, hw)


def _layer2_body(agg_ref, g1_ref, dinv_ref, b1_ref, w2_ref, g2_ref):
    ssum = agg_ref[0] + agg_ref[1] + g1_ref[...]
    h1 = jnp.maximum(ssum * dinv_ref[...] + b1_ref[...], 0.0)
    g2_ref[...] = jnp.dot(h1, w2_ref[...],
                          preferred_element_type=jnp.float32) * dinv_ref[...]


def _layer2(agg1, g1, dinv, b1, w2, rb=1000):
    n, h = g1.shape
    return pl.pallas_call(
        _layer2_body,
        grid=(n // rb,),
        in_specs=[pl.BlockSpec((2, rb, h), lambda i: (0, i, 0)),
                  pl.BlockSpec((rb, h), lambda i: (i, 0)),
                  pl.BlockSpec((rb, 1), lambda i: (i, 0)),
                  pl.BlockSpec((1, h), lambda i: (0, 0)),
                  pl.BlockSpec((h, h), lambda i: (0, 0))],
        out_specs=pl.BlockSpec((rb, h), lambda i: (i, 0)),
        out_shape=jax.ShapeDtypeStruct((n, h), jnp.float32),
    )(agg1, g1, dinv, b1, w2)


def _lstm_stage_body(agg_ref, g2_ref, dinv_ref, b2_ref, wih_t_ref, bsum_ref,
                     whh_t_ref, wl_t_ref, bl_ref, out_ref,
                     h_s, c_s, xw_buf, hs_buf):
    rb = g2_ref.shape[0]
    H = g2_ref.shape[1]

    @pl.when(pl.program_id(0) == 0)
    def _():
        h_s[...] = jnp.zeros_like(h_s)
        c_s[...] = jnp.zeros_like(c_s)

    ssum = agg_ref[0] + agg_ref[1] + g2_ref[...]
    h2 = jnp.maximum(ssum * dinv_ref[...] + b2_ref[...], 0.0)
    xw_buf[...] = jnp.dot(h2, wih_t_ref[...],
                          preferred_element_type=jnp.float32) + bsum_ref[...]

    whh_bf_a = whh_t_ref[0:64, :].astype(jnp.bfloat16)
    whh_bf_b = whh_t_ref[64:128, :].astype(jnp.bfloat16)

    def step(t, carry):
        h, c = carry
        hb = h.astype(jnp.bfloat16)
        gates = (xw_buf[pl.ds(t, 1), :]
                 + jnp.dot(hb[:, 0:64], whh_bf_a,
                           preferred_element_type=jnp.float32)
                 + jnp.dot(hb[:, 64:128], whh_bf_b,
                           preferred_element_type=jnp.float32))
        i = 0.5 + 0.5 * jnp.tanh(0.5 * gates[:, 0:H])
        f = 0.5 + 0.5 * jnp.tanh(0.5 * gates[:, H:2 * H])
        g = jnp.tanh(gates[:, 2 * H:3 * H])
        o = 0.5 + 0.5 * jnp.tanh(0.5 * gates[:, 3 * H:4 * H])
        c2 = f * c + i * g
        h2v = o * jnp.tanh(c2)
        hs_buf[pl.ds(t, 1), :] = h2v
        return (h2v, c2)

    def step8(t8, carry):
        for u in range(8):
            carry = step(t8 * 8 + u, carry)
        return carry

    hN, cN = lax.fori_loop(0, rb // 8, step8, (h_s[...], c_s[...]))
    h_s[...] = hN
    c_s[...] = cN
    out_ref[...] = jnp.dot(hs_buf[...], wl_t_ref[...],
                           preferred_element_type=jnp.float32) + bl_ref[...]


def _lstm_stage(agg2, g2, dinv, b2, wih_t, bsum, whh_t, wl_t, bl, rb=1000):
    n, h = g2.shape
    o = wl_t.shape[1]
    return pl.pallas_call(
        _lstm_stage_body,
        grid=(n // rb,),
        in_specs=[pl.BlockSpec((2, rb, h), lambda i: (0, i, 0)),
                  pl.BlockSpec((rb, h), lambda i: (i, 0)),
                  pl.BlockSpec((rb, 1), lambda i: (i, 0)),
                  pl.BlockSpec((1, h), lambda i: (0, 0)),
                  pl.BlockSpec((h, 4 * h), lambda i: (0, 0)),
                  pl.BlockSpec((1, 4 * h), lambda i: (0, 0)),
                  pl.BlockSpec((h, 4 * h), lambda i: (0, 0)),
                  pl.BlockSpec((h, o), lambda i: (0, 0)),
                  pl.BlockSpec((1, o), lambda i: (0, 0))],
        out_specs=pl.BlockSpec((rb, o), lambda i: (i, 0)),
        out_shape=jax.ShapeDtypeStruct((n, o), jnp.float32),
        scratch_shapes=[pltpu.VMEM((1, h), jnp.float32),
                        pltpu.VMEM((1, h), jnp.float32),
                        pltpu.VMEM((rb, 4 * h), jnp.float32),
                        pltpu.VMEM((rb, h), jnp.float32)],
    )(agg2, g2, dinv, b2, wih_t, bsum, whh_t, wl_t, bl)


def kernel(x, edge_index, W1, b1, W2, b2, W_ih, W_hh, b_ih, b_hh, Wl, bl):
    N, F = x.shape
    E = edge_index.shape[1]
    H = W1.shape[1]
    O = Wl.shape[0]
    NP0 = _pad_rows(N)
    EPW = E // _NW
    GRAN = _K * _NIDX
    EPW_P = ((EPW + GRAN - 1) // GRAN) * GRAN
    CH = EPW_P // _K
    src = jnp.pad(edge_index[0].astype(jnp.int32).reshape(_NW, EPW),
                  ((0, 0), (0, EPW_P - EPW))).reshape(_NW, CH, _K)
    dst = jnp.pad(edge_index[1].astype(jnp.int32).reshape(_NW, EPW),
                  ((0, 0), (0, EPW_P - EPW)),
                  constant_values=NP0 - 1).reshape(_NW, CH, _K)

    NP = _pad_rows(N)
    ones_k = jnp.ones((_K, _DEGW), jnp.float32)
    zeros_deg = jnp.zeros((NP // _NS, _DEGW), jnp.float32)
    zeros_h = jnp.zeros((NP // _NS, H), jnp.float32)

    deg2 = _build_deg(N, CH)(dst, ones_k, zeros_deg).reshape(_NC, NP, _DEGW)[:, :N]
    hw1 = _matmul(x, W1)
    dinv, g1 = _dinv_scale(deg2, hw1)
    agg1 = _build_agg(N, H, CH)(g1, src, dst, zeros_h).reshape(_NC, NP, H)[:, :N]
    g2 = _layer2(agg1, g1, dinv, b1.reshape(1, H), W2)
    agg2 = _build_agg(N, H, CH)(g2, src, dst, zeros_h).reshape(_NC, NP, H)[:, :N]
    out = _lstm_stage(agg2, g2, dinv, b2.reshape(1, H),
                      W_ih.T, (b_ih + b_hh).reshape(1, 4 * H),
                      W_hh.T, Wl.T, bl.reshape(1, O))
    return out


# speedup vs baseline: 1.1438x; 1.1438x over previous
import functools

import jax
import jax.numpy as jnp
from jax import lax
from jax.experimental import pallas as pl
from jax.experimental.pallas import tpu as pltpu
from jax.experimental.pallas import tpu_sc as plsc

_NC = 2
_NS = 16
_NW = _NC * _NS
_DEGW = 128
_K = 80


def _pad_rows(N):
    return (N // 128 + 1) * 128


_NBUF = 2
_NIDX = 4


def _build_deg(N, CH):
    NP = _pad_rows(N)
    ROWS = NP // _NS
    mesh = plsc.VectorSubcoreMesh(core_axis_name="c", subcore_axis_name="s")
    assert CH % _NIDX == 0

    @functools.partial(
        pl.kernel,
        out_type=jax.ShapeDtypeStruct((_NC * NP, _DEGW), jnp.float32),
        mesh=mesh,
        scratch_types=[
            [pltpu.VMEM((_K,), jnp.int32) for _ in range(_NIDX)],
            pltpu.VMEM((_K, _DEGW), jnp.float32),
            pltpu.SemaphoreType.DMA((_NIDX,)),
            pltpu.VMEM_SHARED((NP, _DEGW), jnp.float32),
        ],
    )
    def deg_k(dst_hbm, ones_hbm, zeros_hbm, out_hbm, idx_v, ones_v, dsem,
              acc_sh):
        c = lax.axis_index("c")
        s = lax.axis_index("s")
        w = c * _NS + s
        pltpu.sync_copy(ones_hbm, ones_v)
        pltpu.sync_copy(zeros_hbm, acc_sh.at[pl.ds(s * ROWS, ROWS)])
        for q in range(_NIDX):
            pltpu.async_copy(dst_hbm.at[w, q], idx_v[q], dsem.at[q])
        plsc.subcore_barrier()

        def body(j, carry):
            for r in range(_NIDX):
                i = j * _NIDX + r
                pltpu.make_async_copy(dst_hbm.at[0, 0], idx_v[r],
                                      dsem.at[r]).wait()
                pltpu.sync_copy(ones_v, acc_sh.at[idx_v[r]], add=True)

                @pl.when(i + _NIDX < CH)
                def _():
                    pltpu.async_copy(dst_hbm.at[w, i + _NIDX], idx_v[r],
                                     dsem.at[r])
            return carry

        lax.fori_loop(0, CH // _NIDX, body, 0)
        plsc.subcore_barrier()
        pltpu.sync_copy(acc_sh.at[pl.ds(s * ROWS, ROWS)],
                        out_hbm.at[pl.ds(c * NP + s * ROWS, ROWS)])

    return deg_k


def _build_agg(N, H, CH):
    NP = _pad_rows(N)
    ROWS = NP // _NS
    mesh = plsc.VectorSubcoreMesh(core_axis_name="c", subcore_axis_name="s")
    assert CH % _NIDX == 0

    @functools.partial(
        pl.kernel,
        out_type=jax.ShapeDtypeStruct((_NC * NP, H), jnp.float32),
        mesh=mesh,
        scratch_types=[
            [pltpu.VMEM((_K,), jnp.int32) for _ in range(_NIDX)],
            [pltpu.VMEM((_K,), jnp.int32) for _ in range(_NIDX)],
            [pltpu.VMEM((_K, H), jnp.float32) for _ in range(_NBUF)],
            pltpu.SemaphoreType.DMA((_NIDX,)),
            pltpu.SemaphoreType.DMA((_NIDX,)),
            pltpu.SemaphoreType.DMA((_NBUF,)),
            pltpu.VMEM_SHARED((NP, H), jnp.float32),
        ],
    )
    def agg_k(g_hbm, src_hbm, dst_hbm, zeros_hbm, out_hbm,
              isrc_v, idst_v, rows_v, isem, dsem, gsem, acc_sh):
        c = lax.axis_index("c")
        s = lax.axis_index("s")
        w = c * _NS + s
        pltpu.sync_copy(zeros_hbm, acc_sh.at[pl.ds(s * ROWS, ROWS)])

        for q in range(_NIDX):
            pltpu.async_copy(src_hbm.at[w, q], isrc_v[q], isem.at[q])
            pltpu.async_copy(dst_hbm.at[w, q], idst_v[q], dsem.at[q])
        plsc.subcore_barrier()
        for b in range(_NBUF):
            pltpu.make_async_copy(src_hbm.at[0, 0], isrc_v[b],
                                  isem.at[b]).wait()
            pltpu.async_copy(g_hbm.at[isrc_v[b]], rows_v[b], gsem.at[b])

        def outer(j, carry):
            for r in range(_NIDX):
                i = j * _NIDX + r
                b = r % _NBUF
                pltpu.make_async_copy(g_hbm.at[pl.ds(0, _K)], rows_v[b],
                                      gsem.at[b]).wait()
                pltpu.make_async_copy(src_hbm.at[0, 0], idst_v[r],
                                      dsem.at[r]).wait()
                pltpu.sync_copy(rows_v[b], acc_sh.at[idst_v[r]], add=True)

                @pl.when(i + _NBUF < CH)
                def _():
                    nxt = i + _NBUF
                    pltpu.make_async_copy(src_hbm.at[0, 0],
                                          isrc_v[(r + _NBUF) % _NIDX],
                                          isem.at[(r + _NBUF) % _NIDX]).wait()
                    pltpu.async_copy(g_hbm.at[isrc_v[(r + _NBUF) % _NIDX]],
                                     rows_v[b], gsem.at[b])

                @pl.when(i + _NIDX < CH)
                def _():
                    nn = i + _NIDX
                    pltpu.async_copy(src_hbm.at[w, nn], isrc_v[r],
                                     isem.at[r])
                    pltpu.async_copy(dst_hbm.at[w, nn], idst_v[r],
                                     dsem.at[r])
            return carry

        lax.fori_loop(0, CH // _NIDX, outer, 0)
        plsc.subcore_barrier()
        pltpu.sync_copy(acc_sh.at[pl.ds(s * ROWS, ROWS)],
                        out_hbm.at[pl.ds(c * NP + s * ROWS, ROWS)])

    return agg_k


def _mm_body(x_ref, w_ref, o_ref):
    o_ref[...] = jnp.dot(x_ref[...], w_ref[...],
                         preferred_element_type=jnp.float32)


def _matmul(x, w, rb=1000):
    n, f = x.shape
    h = w.shape[1]
    return pl.pallas_call(
        _mm_body,
        grid=(n // rb,),
        in_specs=[pl.BlockSpec((rb, f), lambda i: (i, 0)),
                  pl.BlockSpec((f, h), lambda i: (0, 0))],
        out_specs=pl.BlockSpec((rb, h), lambda i: (i, 0)),
        out_shape=jax.ShapeDtypeStruct((n, h), jnp.float32),
    )(x, w)


def _dinv_scale_body(deg_ref, hw_ref, dinv_ref, g_ref):
    deg = deg_ref[0, :, 0] + deg_ref[1, :, 0] + 1.0
    dinv = lax.rsqrt(deg)
    dinv_ref[...] = dinv[:, None]
    g_ref[...] = hw_ref[...] * dinv[:, None]


def _dinv_scale(deg2, hw, rb=1000):
    n, h = hw.shape
    return pl.pallas_call(
        _dinv_scale_body,
        grid=(n // rb,),
        in_specs=[pl.BlockSpec((2, rb, _DEGW), lambda i: (0, i, 0)),
                  pl.BlockSpec((rb, h), lambda i: (i, 0))],
        out_specs=[pl.BlockSpec((rb, 1), lambda i: (i, 0)),
                   pl.BlockSpec((rb, h), lambda i: (i, 0))],
        out_shape=[jax.ShapeDtypeStruct((n, 1), jnp.float32),
                   jax.ShapeDtypeStruct((n, h), jnp.float32)],
    )(deg2, hw)


def _layer2_body(agg_ref, g1_ref, dinv_ref, b1_ref, w2_ref, g2_ref):
    ssum = agg_ref[0] + agg_ref[1] + g1_ref[...]
    h1 = jnp.maximum(ssum * dinv_ref[...] + b1_ref[...], 0.0)
    g2_ref[...] = jnp.dot(h1, w2_ref[...],
                          preferred_element_type=jnp.float32) * dinv_ref[...]


def _layer2(agg1, g1, dinv, b1, w2, rb=1000):
    n, h = g1.shape
    return pl.pallas_call(
        _layer2_body,
        grid=(n // rb,),
        in_specs=[pl.BlockSpec((2, rb, h), lambda i: (0, i, 0)),
                  pl.BlockSpec((rb, h), lambda i: (i, 0)),
                  pl.BlockSpec((rb, 1), lambda i: (i, 0)),
                  pl.BlockSpec((1, h), lambda i: (0, 0)),
                  pl.BlockSpec((h, h), lambda i: (0, 0))],
        out_specs=pl.BlockSpec((rb, h), lambda i: (i, 0)),
        out_shape=jax.ShapeDtypeStruct((n, h), jnp.float32),
    )(agg1, g1, dinv, b1, w2)


def _lstm_stage_body(agg_ref, g2_ref, dinv_ref, b2_ref, wih_t_ref, bsum_ref,
                     whh_t_ref, wl_t_ref, bl_ref, out_ref,
                     h_s, c_s, xw_buf, hs_buf):
    rb = g2_ref.shape[0]
    H = g2_ref.shape[1]

    @pl.when(pl.program_id(0) == 0)
    def _():
        h_s[...] = jnp.zeros_like(h_s)
        c_s[...] = jnp.zeros_like(c_s)

    ssum = agg_ref[0] + agg_ref[1] + g2_ref[...]
    h2 = jnp.maximum(ssum * dinv_ref[...] + b2_ref[...], 0.0)
    xw_buf[...] = jnp.dot(h2, wih_t_ref[...],
                          preferred_element_type=jnp.float32) + bsum_ref[...]

    whh_bf = whh_t_ref[...].astype(jnp.bfloat16)

    def step(t, carry):
        h, c = carry
        gates = xw_buf[pl.ds(t, 1), :] + jnp.dot(
            h.astype(jnp.bfloat16), whh_bf,
            preferred_element_type=jnp.float32)
        i = 0.5 + 0.5 * jnp.tanh(0.5 * gates[:, 0:H])
        f = 0.5 + 0.5 * jnp.tanh(0.5 * gates[:, H:2 * H])
        g = jnp.tanh(gates[:, 2 * H:3 * H])
        o = 0.5 + 0.5 * jnp.tanh(0.5 * gates[:, 3 * H:4 * H])
        c2 = f * c + i * g
        h2v = o * jnp.tanh(c2)
        hs_buf[pl.ds(t, 1), :] = h2v
        return (h2v, c2)

    def step8(t8, carry):
        for u in range(8):
            carry = step(t8 * 8 + u, carry)
        return carry

    hN, cN = lax.fori_loop(0, rb // 8, step8, (h_s[...], c_s[...]))
    h_s[...] = hN
    c_s[...] = cN
    out_ref[...] = jnp.dot(hs_buf[...], wl_t_ref[...],
                           preferred_element_type=jnp.float32) + bl_ref[...]


def _lstm_stage(agg2, g2, dinv, b2, wih_t, bsum, whh_t, wl_t, bl, rb=1000):
    n, h = g2.shape
    o = wl_t.shape[1]
    return pl.pallas_call(
        _lstm_stage_body,
        grid=(n // rb,),
        in_specs=[pl.BlockSpec((2, rb, h), lambda i: (0, i, 0)),
                  pl.BlockSpec((rb, h), lambda i: (i, 0)),
                  pl.BlockSpec((rb, 1), lambda i: (i, 0)),
                  pl.BlockSpec((1, h), lambda i: (0, 0)),
                  pl.BlockSpec((h, 4 * h), lambda i: (0, 0)),
                  pl.BlockSpec((1, 4 * h), lambda i: (0, 0)),
                  pl.BlockSpec((h, 4 * h), lambda i: (0, 0)),
                  pl.BlockSpec((h, o), lambda i: (0, 0)),
                  pl.BlockSpec((1, o), lambda i: (0, 0))],
        out_specs=pl.BlockSpec((rb, o), lambda i: (i, 0)),
        out_shape=jax.ShapeDtypeStruct((n, o), jnp.float32),
        scratch_shapes=[pltpu.VMEM((1, h), jnp.float32),
                        pltpu.VMEM((1, h), jnp.float32),
                        pltpu.VMEM((rb, 4 * h), jnp.float32),
                        pltpu.VMEM((rb, h), jnp.float32)],
    )(agg2, g2, dinv, b2, wih_t, bsum, whh_t, wl_t, bl)


def kernel(x, edge_index, W1, b1, W2, b2, W_ih, W_hh, b_ih, b_hh, Wl, bl):
    N, F = x.shape
    E = edge_index.shape[1]
    H = W1.shape[1]
    O = Wl.shape[0]
    NP0 = _pad_rows(N)
    EPW = E // _NW
    GRAN = _K * _NIDX
    EPW_P = ((EPW + GRAN - 1) // GRAN) * GRAN
    CH = EPW_P // _K
    src = jnp.pad(edge_index[0].astype(jnp.int32).reshape(_NW, EPW),
                  ((0, 0), (0, EPW_P - EPW))).reshape(_NW, CH, _K)
    dst = jnp.pad(edge_index[1].astype(jnp.int32).reshape(_NW, EPW),
                  ((0, 0), (0, EPW_P - EPW)),
                  constant_values=NP0 - 1).reshape(_NW, CH, _K)

    NP = _pad_rows(N)
    ones_k = jnp.ones((_K, _DEGW), jnp.float32)
    zeros_deg = jnp.zeros((NP // _NS, _DEGW), jnp.float32)
    zeros_h = jnp.zeros((NP // _NS, H), jnp.float32)

    deg2 = _build_deg(N, CH)(dst, ones_k, zeros_deg).reshape(_NC, NP, _DEGW)[:, :N]
    hw1 = _matmul(x, W1)
    dinv, g1 = _dinv_scale(deg2, hw1)
    agg1 = _build_agg(N, H, CH)(g1, src, dst, zeros_h).reshape(_NC, NP, H)[:, :N]
    g2 = _layer2(agg1, g1, dinv, b1.reshape(1, H), W2)
    agg2 = _build_agg(N, H, CH)(g2, src, dst, zeros_h).reshape(_NC, NP, H)[:, :N]
    out = _lstm_stage(agg2, g2, dinv, b2.reshape(1, H),
                      W_ih.T, (b_ih + b_hh).reshape(1, 4 * H),
                      W_hh.T, Wl.T, bl.reshape(1, O))
    return out


# speedup vs baseline: 1.3134x; 1.1483x over previous
import functools

import jax
import jax.numpy as jnp
from jax import lax
from jax.experimental import pallas as pl
from jax.experimental.pallas import tpu as pltpu
from jax.experimental.pallas import tpu_sc as plsc

_NC = 2
_NS = 16
_NW = _NC * _NS
_DEGW = 128
_K = 40


def _pad_rows(N):
    return (N // 128 + 1) * 128


_NBUF = 2
_NIDX = 4


def _build_deg(N, CH):
    NP = _pad_rows(N)
    ROWS = NP // _NS
    mesh = plsc.VectorSubcoreMesh(core_axis_name="c", subcore_axis_name="s")
    assert CH % _NIDX == 0

    @functools.partial(
        pl.kernel,
        out_type=jax.ShapeDtypeStruct((_NC * NP, _DEGW), jnp.float32),
        mesh=mesh,
        scratch_types=[
            [pltpu.VMEM((_K,), jnp.int32) for _ in range(_NIDX)],
            pltpu.VMEM((_K, _DEGW), jnp.float32),
            pltpu.SemaphoreType.DMA((_NIDX,)),
            pltpu.VMEM_SHARED((NP, _DEGW), jnp.float32),
        ],
    )
    def deg_k(dst_hbm, ones_hbm, zeros_hbm, out_hbm, idx_v, ones_v, dsem,
              acc_sh):
        c = lax.axis_index("c")
        s = lax.axis_index("s")
        w = c * _NS + s
        pltpu.sync_copy(ones_hbm, ones_v)
        pltpu.sync_copy(zeros_hbm, acc_sh.at[pl.ds(s * ROWS, ROWS)])
        for q in range(_NIDX):
            pltpu.async_copy(dst_hbm.at[w, q], idx_v[q], dsem.at[q])
        plsc.subcore_barrier()

        def body(j, carry):
            for r in range(_NIDX):
                i = j * _NIDX + r
                pltpu.make_async_copy(dst_hbm.at[0, 0], idx_v[r],
                                      dsem.at[r]).wait()
                pltpu.sync_copy(ones_v, acc_sh.at[idx_v[r]], add=True)

                @pl.when(i + _NIDX < CH)
                def _():
                    pltpu.async_copy(dst_hbm.at[w, i + _NIDX], idx_v[r],
                                     dsem.at[r])
            return carry

        lax.fori_loop(0, CH // _NIDX, body, 0)
        plsc.subcore_barrier()
        pltpu.sync_copy(acc_sh.at[pl.ds(s * ROWS, ROWS)],
                        out_hbm.at[pl.ds(c * NP + s * ROWS, ROWS)])

    return deg_k


def _build_agg(N, H, CH):
    NP = _pad_rows(N)
    ROWS = NP // _NS
    mesh = plsc.VectorSubcoreMesh(core_axis_name="c", subcore_axis_name="s")
    assert CH % _NIDX == 0

    @functools.partial(
        pl.kernel,
        out_type=jax.ShapeDtypeStruct((_NC * NP, H), jnp.float32),
        mesh=mesh,
        scratch_types=[
            [pltpu.VMEM((_K,), jnp.int32) for _ in range(_NIDX)],
            [pltpu.VMEM((_K,), jnp.int32) for _ in range(_NIDX)],
            [pltpu.VMEM((_K, H), jnp.float32) for _ in range(_NBUF)],
            pltpu.SemaphoreType.DMA((_NIDX,)),
            pltpu.SemaphoreType.DMA((_NIDX,)),
            pltpu.SemaphoreType.DMA((_NBUF,)),
            pltpu.VMEM_SHARED((NP, H), jnp.float32),
        ],
    )
    def agg_k(g_hbm, src_hbm, dst_hbm, zeros_hbm, out_hbm,
              isrc_v, idst_v, rows_v, isem, dsem, gsem, acc_sh):
        c = lax.axis_index("c")
        s = lax.axis_index("s")
        w = c * _NS + s
        pltpu.sync_copy(zeros_hbm, acc_sh.at[pl.ds(s * ROWS, ROWS)])

        for q in range(_NIDX):
            pltpu.async_copy(src_hbm.at[w, q], isrc_v[q], isem.at[q])
            pltpu.async_copy(dst_hbm.at[w, q], idst_v[q], dsem.at[q])
        plsc.subcore_barrier()
        for b in range(_NBUF):
            pltpu.make_async_copy(src_hbm.at[0, 0], isrc_v[b],
                                  isem.at[b]).wait()
            pltpu.async_copy(g_hbm.at[isrc_v[b]], rows_v[b], gsem.at[b])

        def outer(j, carry):
            for r in range(_NIDX):
                i = j * _NIDX + r
                b = r % _NBUF
                pltpu.make_async_copy(g_hbm.at[pl.ds(0, _K)], rows_v[b],
                                      gsem.at[b]).wait()
                pltpu.make_async_copy(src_hbm.at[0, 0], idst_v[r],
                                      dsem.at[r]).wait()
                pltpu.sync_copy(rows_v[b], acc_sh.at[idst_v[r]], add=True)

                @pl.when(i + _NBUF < CH)
                def _():
                    nxt = i + _NBUF
                    pltpu.make_async_copy(src_hbm.at[0, 0],
                                          isrc_v[(r + _NBUF) % _NIDX],
                                          isem.at[(r + _NBUF) % _NIDX]).wait()
                    pltpu.async_copy(g_hbm.at[isrc_v[(r + _NBUF) % _NIDX]],
                                     rows_v[b], gsem.at[b])

                @pl.when(i + _NIDX < CH)
                def _():
                    nn = i + _NIDX
                    pltpu.async_copy(src_hbm.at[w, nn], isrc_v[r],
                                     isem.at[r])
                    pltpu.async_copy(dst_hbm.at[w, nn], idst_v[r],
                                     dsem.at[r])
            return carry

        lax.fori_loop(0, CH // _NIDX, outer, 0)
        plsc.subcore_barrier()
        pltpu.sync_copy(acc_sh.at[pl.ds(s * ROWS, ROWS)],
                        out_hbm.at[pl.ds(c * NP + s * ROWS, ROWS)])

    return agg_k


def _mm_body(x_ref, w_ref, o_ref):
    o_ref[...] = jnp.dot(x_ref[...], w_ref[...],
                         preferred_element_type=jnp.float32)


def _matmul(x, w, rb=1000):
    n, f = x.shape
    h = w.shape[1]
    return pl.pallas_call(
        _mm_body,
        grid=(n // rb,),
        in_specs=[pl.BlockSpec((rb, f), lambda i: (i, 0)),
                  pl.BlockSpec((f, h), lambda i: (0, 0))],
        out_specs=pl.BlockSpec((rb, h), lambda i: (i, 0)),
        out_shape=jax.ShapeDtypeStruct((n, h), jnp.float32),
    )(x, w)


def _dinv_scale_body(deg_ref, hw_ref, dinv_ref, g_ref):
    deg = deg_ref[0, :, 0] + deg_ref[1, :, 0] + 1.0
    dinv = lax.rsqrt(deg)
    dinv_ref[...] = dinv[:, None]
    g_ref[...] = hw_ref[...] * dinv[:, None]


def _dinv_scale(deg2, hw, rb=1000):
    n, h = hw.shape
    return pl.pallas_call(
        _dinv_scale_body,
        grid=(n // rb,),
        in_specs=[pl.BlockSpec((2, rb, _DEGW), lambda i: (0, i, 0)),
                  pl.BlockSpec((rb, h), lambda i: (i, 0))],
        out_specs=[pl.BlockSpec((rb, 1), lambda i: (i, 0)),
                   pl.BlockSpec((rb, h), lambda i: (i, 0))],
        out_shape=[jax.ShapeDtypeStruct((n, 1), jnp.float32),
                   jax.ShapeDtypeStruct((n, h), jnp.float32)],
    )(deg2, hw)


def _layer2_body(agg_ref, g1_ref, dinv_ref, b1_ref, w2_ref, g2_ref):
    ssum = agg_ref[0] + agg_ref[1] + g1_ref[...]
    h1 = jnp.maximum(ssum * dinv_ref[...] + b1_ref[...], 0.0)
    g2_ref[...] = jnp.dot(h1, w2_ref[...],
                          preferred_element_type=jnp.float32) * dinv_ref[...]


def _layer2(agg1, g1, dinv, b1, w2, rb=1000):
    n, h = g1.shape
    return pl.pallas_call(
        _layer2_body,
        grid=(n // rb,),
        in_specs=[pl.BlockSpec((2, rb, h), lambda i: (0, i, 0)),
                  pl.BlockSpec((rb, h), lambda i: (i, 0)),
                  pl.BlockSpec((rb, 1), lambda i: (i, 0)),
                  pl.BlockSpec((1, h), lambda i: (0, 0)),
                  pl.BlockSpec((h, h), lambda i: (0, 0))],
        out_specs=pl.BlockSpec((rb, h), lambda i: (i, 0)),
        out_shape=jax.ShapeDtypeStruct((n, h), jnp.float32),
    )(agg1, g1, dinv, b1, w2)


def _lstm_stage_body(agg_ref, g2_ref, dinv_ref, b2_ref, wih_t_ref, bsum_ref,
                     whh_t_ref, wl_t_ref, bl_ref, out_ref,
                     h_s, c_s, xw_buf, hs_buf):
    rb = g2_ref.shape[0]
    H = g2_ref.shape[1]

    @pl.when(pl.program_id(0) == 0)
    def _():
        h_s[...] = jnp.zeros_like(h_s)
        c_s[...] = jnp.zeros_like(c_s)

    ssum = agg_ref[0] + agg_ref[1] + g2_ref[...]
    h2 = jnp.maximum(ssum * dinv_ref[...] + b2_ref[...], 0.0)
    xw_buf[...] = jnp.dot(h2, wih_t_ref[...],
                          preferred_element_type=jnp.float32) + bsum_ref[...]

    whh_bf = whh_t_ref[...].astype(jnp.bfloat16)

    def step(t, carry):
        h, c = carry
        gates = xw_buf[pl.ds(t, 1), :] + jnp.dot(
            h.astype(jnp.bfloat16), whh_bf,
            preferred_element_type=jnp.float32)
        i = 0.5 + 0.5 * jnp.tanh(0.5 * gates[:, 0:H])
        f = 0.5 + 0.5 * jnp.tanh(0.5 * gates[:, H:2 * H])
        g = jnp.tanh(gates[:, 2 * H:3 * H])
        o = 0.5 + 0.5 * jnp.tanh(0.5 * gates[:, 3 * H:4 * H])
        c2 = f * c + i * g
        h2v = o * jnp.tanh(c2)
        hs_buf[pl.ds(t, 1), :] = h2v
        return (h2v, c2)

    def step8(t8, carry):
        for u in range(8):
            carry = step(t8 * 8 + u, carry)
        return carry

    hN, cN = lax.fori_loop(0, rb // 8, step8, (h_s[...], c_s[...]))
    h_s[...] = hN
    c_s[...] = cN
    out_ref[...] = jnp.dot(hs_buf[...], wl_t_ref[...],
                           preferred_element_type=jnp.float32) + bl_ref[...]


def _lstm_stage(agg2, g2, dinv, b2, wih_t, bsum, whh_t, wl_t, bl, rb=1000):
    n, h = g2.shape
    o = wl_t.shape[1]
    return pl.pallas_call(
        _lstm_stage_body,
        grid=(n // rb,),
        in_specs=[pl.BlockSpec((2, rb, h), lambda i: (0, i, 0)),
                  pl.BlockSpec((rb, h), lambda i: (i, 0)),
                  pl.BlockSpec((rb, 1), lambda i: (i, 0)),
                  pl.BlockSpec((1, h), lambda i: (0, 0)),
                  pl.BlockSpec((h, 4 * h), lambda i: (0, 0)),
                  pl.BlockSpec((1, 4 * h), lambda i: (0, 0)),
                  pl.BlockSpec((h, 4 * h), lambda i: (0, 0)),
                  pl.BlockSpec((h, o), lambda i: (0, 0)),
                  pl.BlockSpec((1, o), lambda i: (0, 0))],
        out_specs=pl.BlockSpec((rb, o), lambda i: (i, 0)),
        out_shape=jax.ShapeDtypeStruct((n, o), jnp.float32),
        scratch_shapes=[pltpu.VMEM((1, h), jnp.float32),
                        pltpu.VMEM((1, h), jnp.float32),
                        pltpu.VMEM((rb, 4 * h), jnp.float32),
                        pltpu.VMEM((rb, h), jnp.float32)],
    )(agg2, g2, dinv, b2, wih_t, bsum, whh_t, wl_t, bl)


def kernel(x, edge_index, W1, b1, W2, b2, W_ih, W_hh, b_ih, b_hh, Wl, bl):
    N, F = x.shape
    E = edge_index.shape[1]
    H = W1.shape[1]
    O = Wl.shape[0]
    NP0 = _pad_rows(N)
    EPW = E // _NW
    GRAN = _K * _NIDX
    EPW_P = ((EPW + GRAN - 1) // GRAN) * GRAN
    CH = EPW_P // _K
    src = jnp.pad(edge_index[0].astype(jnp.int32).reshape(_NW, EPW),
                  ((0, 0), (0, EPW_P - EPW))).reshape(_NW, CH, _K)
    dst = jnp.pad(edge_index[1].astype(jnp.int32).reshape(_NW, EPW),
                  ((0, 0), (0, EPW_P - EPW)),
                  constant_values=NP0 - 1).reshape(_NW, CH, _K)

    NP = _pad_rows(N)
    ones_k = jnp.ones((_K, _DEGW), jnp.float32)
    zeros_deg = jnp.zeros((NP // _NS, _DEGW), jnp.float32)
    zeros_h = jnp.zeros((NP // _NS, H), jnp.float32)

    deg2 = _build_deg(N, CH)(dst, ones_k, zeros_deg).reshape(_NC, NP, _DEGW)[:, :N]
    hw1 = _matmul(x, W1)
    dinv, g1 = _dinv_scale(deg2, hw1)
    agg1 = _build_agg(N, H, CH)(g1, src, dst, zeros_h).reshape(_NC, NP, H)[:, :N]
    g2 = _layer2(agg1, g1, dinv, b1.reshape(1, H), W2)
    agg2 = _build_agg(N, H, CH)(g2, src, dst, zeros_h).reshape(_NC, NP, H)[:, :N]
    out = _lstm_stage(agg2, g2, dinv, b2.reshape(1, H),
                      W_ih.T, (b_ih + b_hh).reshape(1, 4 * H),
                      W_hh.T, Wl.T, bl.reshape(1, O))
    return out
